# baseline (speedup 1.0000x reference)
import functools

import jax
import jax.numpy as jnp
from jax import lax
from jax.experimental import pallas as pl
from jax.experimental.pallas import tpu as pltpu
from jax.experimental.pallas import tpu_sc as plsc

N_NODES = 100000
N_FOCAL = 25000
D_FEAT = 128
NK = 8

NW = 32
CHUNK = 3200
PAD_BLK = 25600
NE = NW * CHUNK
NB = 100352
PAD_KEY = N_NODES

S_SPLIT = 50048
RBLK = 128
NREAL = S_SPLIT // RBLK
NZERO = 2
NGRID = NREAL + NZERO
PKROWS = NGRID * RBLK
TROWS = PKROWS * 8
FILL_BASE = 2 * S_SPLIT


def _proj_body(xa_ref, xb_ref, w_ref, o_ref):
    i = pl.program_id(0)
    lane = lax.broadcasted_iota(jnp.int32, (1, 128), 1)
    v = lane & 15
    ddv = (lane >> 4) & 3
    scale = jnp.where(v < 8, 1.0, 1.0 / (ddv + 1).astype(jnp.float32))
    wmat = w_ref[...] * scale
    a = jnp.concatenate([xa_ref[...], xb_ref[...]], axis=1)
    res = lax.dot_general(a, wmat, (((1,), (0,)), ((), ())),
                          preferred_element_type=jnp.float32)
    o_ref[...] = jnp.where(i >= NREAL, 0.0, res)


def _project(x, wbig):
    return pl.pallas_call(
        _proj_body,
        grid=(NGRID,),
        in_specs=[
            pl.BlockSpec((RBLK, D_FEAT), lambda i: (jnp.minimum(i, NREAL - 1), 0)),
            pl.BlockSpec((RBLK, D_FEAT),
                         lambda i: (jnp.minimum(NREAL + i, (N_NODES - 1) // RBLK), 0)),
            pl.BlockSpec((2 * D_FEAT, 128), lambda i: (0, 0)),
        ],
        out_specs=pl.BlockSpec((RBLK, 128), lambda i: (i, 0)),
        out_shape=jax.ShapeDtypeStruct((PKROWS, 128), jnp.float32),
    )(x, x, wbig)


def _offsets_body(h_ref, t_ref, carry_ref):
    pid = pl.program_id(0)

    @pl.when(pid == 0)
    def _():
        carry_ref[0] = 0

    blk = h_ref[...]

    def shift_down0(a, s):
        return jnp.pad(a, ((s, 0), (0, 0)))[: a.shape[0], :]

    def shift_down1(a, s):
        return jnp.pad(a, ((0, 0), (s, 0)))[:, : a.shape[1]]

    cum0 = blk
    s = 1
    while s < 32:
        cum0 = cum0 + shift_down0(cum0, s)
        s *= 2
    wexcl = cum0 - blk

    total = jnp.sum(blk, axis=0, keepdims=True)
    cum1 = total
    s = 1
    while s < 2048:
        cum1 = cum1 + shift_down1(cum1, s)
        s *= 2
    carry = carry_ref[0]
    excl_bins = cum1 - total + carry
    t_ref[...] = wexcl + excl_bins
    carry_ref[0] = carry + jnp.sum(total)


def _offsets(hgrid):
    nblk = NB // 2048
    return pl.pallas_call(
        _offsets_body,
        grid=(nblk,),
        in_specs=[pl.BlockSpec((NW, 2048), lambda i: (0, i))],
        out_specs=pl.BlockSpec((NW, 2048), lambda i: (0, i)),
        out_shape=jax.ShapeDtypeStruct((NW, NB), jnp.int32),
        scratch_shapes=[pltpu.SMEM((1,), jnp.int32)],
    )(hgrid)


_MESH = plsc.VectorSubcoreMesh(core_axis_name="c", subcore_axis_name="s")


def _wid():
    return lax.axis_index("s") * 2 + lax.axis_index("c")


@functools.partial(
    pl.kernel,
    mesh=_MESH,
    compiler_params=pltpu.CompilerParams(needs_layout_passes=False),
    out_type=(
        jax.ShapeDtypeStruct((NW * NB,), jnp.int32),
        jax.ShapeDtypeStruct((NE,), jnp.int32),
    ),
    scratch_types=[
        pltpu.VMEM((NB,), jnp.int32),
        pltpu.VMEM((CHUNK,), jnp.int32),
        pltpu.VMEM((CHUNK,), jnp.int32),
        pltpu.VMEM((48,), jnp.int32),
    ],
)
def _k1(keys_hbm, h_hbm, rank_hbm, hist, keysb, rankb, shf):
    w = _wid()
    pltpu.sync_copy(keys_hbm.at[pl.ds(w * CHUNK, CHUNK)], keysb)

    zero16 = jnp.zeros((16,), jnp.int32)

    def zbody(j, c):
        hist[pl.ds(j * 16, 16)] = zero16
        return c

    lax.fori_loop(0, NB // 16, zbody, 0)

    neg16 = jnp.full((16,), -1, jnp.int32)
    shf[pl.ds(0, 16)] = neg16
    shf[pl.ds(16, 16)] = neg16
    shf[pl.ds(32, 16)] = neg16

    def body(i, c):
        off = i * 16
        kv = keysb[pl.ds(off, 16)]
        shf[pl.ds(15, 16)] = kv
        within = jnp.zeros((16,), jnp.int32)
        after = jnp.zeros((16,), jnp.int32)
        for k in range(1, 16):
            lv = shf[pl.ds(15 - k, 16)]
            within = within + jnp.where(lv == kv, 1, 0)
            rv = shf[pl.ds(15 + k, 16)]
            after = after + jnp.where(rv == kv, 1, 0)
        rb = plsc.load_gather(hist, [kv])
        rankb[pl.ds(off, 16)] = rb + within
        plsc.store_scatter(hist, [kv], rb + within + 1, mask=after == 0)
        return c

    lax.fori_loop(0, CHUNK // 16, body, 0)

    pltpu.sync_copy(hist, h_hbm.at[pl.ds(w * NB, NB)])
    pltpu.sync_copy(rankb, rank_hbm.at[pl.ds(w * CHUNK, CHUNK)])


@functools.partial(
    pl.kernel,
    mesh=_MESH,
    compiler_params=pltpu.CompilerParams(needs_layout_passes=False),
    out_type=jax.ShapeDtypeStruct((NE,), jnp.int32),
    scratch_types=[
        pltpu.VMEM((NB,), jnp.int32),
        pltpu.VMEM((CHUNK,), jnp.int32),
        pltpu.VMEM((CHUNK,), jnp.int32),
        pltpu.VMEM((CHUNK,), jnp.int32),
    ],
)
def _k2a(t_hbm, keys_hbm, rank_hbm, pos_hbm, tb, kb, rb, pb):
    w = _wid()
    pltpu.sync_copy(t_hbm.at[pl.ds(w * NB, NB)], tb)
    pltpu.sync_copy(keys_hbm.at[pl.ds(w * CHUNK, CHUNK)], kb)
    pltpu.sync_copy(rank_hbm.at[pl.ds(w * CHUNK, CHUNK)], rb)

    def body(i, c):
        off = i * 16
        kv = kb[pl.ds(off, 16)]
        tv = plsc.load_gather(tb, [kv])
        pb[pl.ds(off, 16)] = tv + rb[pl.ds(off, 16)]
        return c

    lax.fori_loop(0, CHUNK // 16, body, 0)
    pltpu.sync_copy(pb, pos_hbm.at[pl.ds(w * CHUNK, CHUNK)])


_NSUB = CHUNK // 128


@functools.partial(
    pl.kernel,
    mesh=_MESH,
    compiler_params=pltpu.CompilerParams(needs_layout_passes=False,
                                         use_tc_tiling_on_sc=False),
    out_type=jax.ShapeDtypeStruct((NE, 16), jnp.float32),
    scratch_types=[
        pltpu.VMEM((CHUNK,), jnp.int32),
        pltpu.VMEM((4 * CHUNK,), jnp.int32),
        pltpu.VMEM((4 * CHUNK,), jnp.int32),
        pltpu.VMEM((_NSUB, 128), jnp.int32),
        pltpu.VMEM((128, 16), jnp.float32),
        pltpu.VMEM((512, 16), jnp.float32),
        pltpu.VMEM((128, 16), jnp.float32),
        pltpu.VMEM((32,), jnp.float32),
        pltpu.SemaphoreType.DMA,
        pltpu.SemaphoreType.DMA,
    ],
)
def _k2b(pcat_hbm, sel_hbm, nei1_hbm, nei2_hbm, nei3_hbm, nei4_hbm, pos_hbm,
         out_hbm, selb, neib, ub, posb, fb, nb, ob, shf, sem_g, sem_s):
    w = _wid()
    dd = w // 8
    j = w % 8
    pltpu.sync_copy(sel_hbm.at[pl.ds(w * CHUNK, CHUNK)], selb)
    pltpu.sync_copy(pos_hbm.at[w], posb)
    for kd, nref in ((0, nei1_hbm), (1, nei2_hbm), (2, nei3_hbm), (3, nei4_hbm)):
        @pl.when(dd == kd)
        def _(nref=nref, kd=kd):
            ln = CHUNK * (kd + 1)
            pltpu.sync_copy(nref.at[pl.ds(j * ln, ln)], neib.at[pl.ds(0, ln)])

    def to_row(n):
        q = jnp.where(n >= S_SPLIT, 1, 0)
        return (n - q * S_SPLIT) * 8 + q * 4 + dd

    def tsel(i, c):
        off = i * 16
        selb[pl.ds(off, 16)] = to_row(selb[pl.ds(off, 16)])
        return c

    lax.fori_loop(0, CHUNK // 16, tsel, 0)

    iota16 = lax.broadcasted_iota(jnp.int32, (16,), 0)

    def tuni(i, c):
        u0 = i * 16
        uv = u0 + iota16
        el = uv >> 2
        kk = uv & 3
        m = el * (dd + 1) + kk
        raw = plsc.load_gather(neib, [m])
        fillv = FILL_BASE + ((w * (4 * CHUNK) + uv) & 255)
        mg = jnp.where(kk <= dd, raw, fillv)
        ub[pl.ds(u0, 16)] = to_row(mg)
        return c

    lax.fori_loop(0, 4 * CHUNK // 16, tuni, 0)

    shf[pl.ds(16, 16)] = jnp.zeros((16,), jnp.float32)

    def sub(s, c):
        cg = pltpu.async_copy(pcat_hbm.at[selb.at[pl.ds(s * 128, 128)]], fb, sem_g)
        cn0 = pltpu.async_copy(pcat_hbm.at[ub.at[pl.ds(s * 512, 128)]],
                               nb.at[pl.ds(0, 128)], sem_g)
        cn1 = pltpu.async_copy(pcat_hbm.at[ub.at[pl.ds(s * 512 + 128, 128)]],
                               nb.at[pl.ds(128, 128)], sem_g)
        cn2 = pltpu.async_copy(pcat_hbm.at[ub.at[pl.ds(s * 512 + 256, 128)]],
                               nb.at[pl.ds(256, 128)], sem_g)
        cn3 = pltpu.async_copy(pcat_hbm.at[ub.at[pl.ds(s * 512 + 384, 128)]],
                               nb.at[pl.ds(384, 128)], sem_g)
        cg.wait()
        cn0.wait()
        cn1.wait()
        cn2.wait()
        cn3.wait()

        def ebody(e, c2):
            base = e * 4
            acc = (nb[base] + nb[base + 1]) + (nb[base + 2] + nb[base + 3])
            shf[pl.ds(0, 16)] = acc
            sh = shf[pl.ds(8, 16)]
            ob[e] = fb[e] + sh
            return c2

        lax.fori_loop(0, 128, ebody, 0)

        cs = pltpu.async_copy(ob, out_hbm.at[posb.at[s]], sem_s)
        cs.wait()
        return c

    lax.fori_loop(0, _NSUB, sub, 0)


def kernel(is_last_layer, x, edge_index, edge_attr, p,
           p_focal_deg1, p_focal_deg2, p_focal_deg3, p_focal_deg4,
           nei_p_deg1, nei_p_deg2, nei_p_deg3, nei_p_deg4,
           nei_edge_attr_deg1, nei_edge_attr_deg2, nei_edge_attr_deg3, nei_edge_attr_deg4,
           selected_index_deg1, selected_index_deg2, selected_index_deg3, selected_index_deg4,
           nei_index_deg1, nei_index_deg2, nei_index_deg3, nei_index_deg4,
           save_score, W1, W2, W3, W4):
    sels = [selected_index_deg1, selected_index_deg2,
            selected_index_deg3, selected_index_deg4]
    neis = [nei_index_deg1, nei_index_deg2, nei_index_deg3, nei_index_deg4]

    wall = jnp.stack([w.astype(jnp.float32) for w in (W1, W2, W3, W4)])
    cc = jnp.arange(128)
    vv, uu = cc % 16, cc // 16
    ddc, pc = uu % 4, uu // 4
    kk = jnp.arange(2 * D_FEAT)
    krow, pk = kk % D_FEAT, kk // D_FEAT
    wbig = jnp.where(
        (pk[:, None] == pc[None, :]),
        wall[ddc[None, :], krow[:, None] + D_FEAT * (vv[None, :] >= 8), vv[None, :] % 8],
        0.0,
    )
    pcat_packed = _project(x.astype(jnp.float32), wbig)
    table = pcat_packed.reshape(TROWS, 16)

    pad_k = jnp.full((PAD_BLK - N_FOCAL,), PAD_KEY, jnp.int32)
    keys = jnp.concatenate(
        [jnp.concatenate([s.astype(jnp.int32), pad_k]) for s in sels])

    pad_z = jnp.zeros((PAD_BLK - N_FOCAL,), jnp.int32)
    sel_raw = jnp.concatenate(
        [jnp.concatenate([s.astype(jnp.int32), pad_z]) for s in sels])

    nei_pads = [
        jnp.pad(neis[d - 1].astype(jnp.int32), (0, (PAD_BLK - N_FOCAL) * d))
        for d in range(1, 5)
    ]

    hflat, rank = _k1(keys)
    tgrid = _offsets(hflat.reshape(NW, NB))
    pos = _k2a(tgrid.reshape(NW * NB), keys, rank)
    out_pad = _k2b(table, sel_raw, *nei_pads, pos.reshape(NW, _NSUB, 128))
    return out_pad[:N_NODES, :NK]


# speedup vs baseline: 2.0752x; 2.0752x over previous
import functools

import jax
import jax.numpy as jnp
from jax import lax
from jax.experimental import pallas as pl
from jax.experimental.pallas import tpu as pltpu
from jax.experimental.pallas import tpu_sc as plsc

N_NODES = 100000
N_FOCAL = 25000
D_FEAT = 128
NK = 8

NW = 32
CHUNK = 3200
PAD_BLK = 25600
NE = NW * CHUNK
NB = 100352
PAD_KEY = N_NODES

S_SPLIT = 50176
RBLK = 512
NREAL = S_SPLIT // RBLK
NZERO = 1
NGRID = NREAL + NZERO
PKROWS = NGRID * RBLK
TROWS = PKROWS * 8
FILL_BASE = 2 * S_SPLIT


def _proj_body(xa_ref, xb_ref, w_ref, o_ref):
    i = pl.program_id(0)
    lane = lax.broadcasted_iota(jnp.int32, (1, 128), 1)
    v = lane & 15
    ddv = (lane >> 4) & 3
    scale = jnp.where(v < 8, 1.0, 1.0 / (ddv + 1).astype(jnp.float32))
    wmat = w_ref[...] * scale
    a = jnp.concatenate([xa_ref[...], xb_ref[...]], axis=1)
    res = lax.dot_general(a, wmat, (((1,), (0,)), ((), ())),
                          preferred_element_type=jnp.float32)
    o_ref[...] = jnp.where(i >= NREAL, 0.0, res)


def _project(x, wbig):
    return pl.pallas_call(
        _proj_body,
        grid=(NGRID,),
        in_specs=[
            pl.BlockSpec((RBLK, D_FEAT), lambda i: (jnp.minimum(i, NREAL - 1), 0)),
            pl.BlockSpec((RBLK, D_FEAT),
                         lambda i: (jnp.minimum(NREAL + i, (N_NODES - 1) // RBLK), 0)),
            pl.BlockSpec((2 * D_FEAT, 128), lambda i: (0, 0)),
        ],
        out_specs=pl.BlockSpec((RBLK, 128), lambda i: (i, 0)),
        out_shape=jax.ShapeDtypeStruct((PKROWS, 128), jnp.float32),
    )(x, x, wbig)


def _offsets_body(h_ref, t_ref, carry_ref):
    pid = pl.program_id(0)

    @pl.when(pid == 0)
    def _():
        carry_ref[0] = 0

    blk = h_ref[...]

    def shift_down0(a, s):
        return jnp.pad(a, ((s, 0), (0, 0)))[: a.shape[0], :]

    def shift_down1(a, s):
        return jnp.pad(a, ((0, 0), (s, 0)))[:, : a.shape[1]]

    cum0 = blk
    s = 1
    while s < 32:
        cum0 = cum0 + shift_down0(cum0, s)
        s *= 2
    wexcl = cum0 - blk

    total = jnp.sum(blk, axis=0, keepdims=True)
    cum1 = total
    s = 1
    while s < 2048:
        cum1 = cum1 + shift_down1(cum1, s)
        s *= 2
    carry = carry_ref[0]
    excl_bins = cum1 - total + carry
    t_ref[...] = wexcl + excl_bins
    carry_ref[0] = carry + jnp.sum(total)


def _offsets(hgrid):
    nblk = NB // 2048
    return pl.pallas_call(
        _offsets_body,
        grid=(nblk,),
        in_specs=[pl.BlockSpec((NW, 2048), lambda i: (0, i))],
        out_specs=pl.BlockSpec((NW, 2048), lambda i: (0, i)),
        out_shape=jax.ShapeDtypeStruct((NW, NB), jnp.int32),
        scratch_shapes=[pltpu.SMEM((1,), jnp.int32)],
    )(hgrid)


_MESH = plsc.VectorSubcoreMesh(core_axis_name="c", subcore_axis_name="s")


def _wid():
    return lax.axis_index("s") * 2 + lax.axis_index("c")


@functools.partial(
    pl.kernel,
    mesh=_MESH,
    compiler_params=pltpu.CompilerParams(needs_layout_passes=False),
    out_type=(
        jax.ShapeDtypeStruct((NW * NB,), jnp.int32),
        jax.ShapeDtypeStruct((NE,), jnp.int32),
    ),
    scratch_types=[
        pltpu.VMEM((NB,), jnp.int32),
        pltpu.VMEM((CHUNK,), jnp.int32),
        pltpu.VMEM((CHUNK,), jnp.int32),
        pltpu.VMEM((48,), jnp.int32),
    ],
)
def _k1(keys_hbm, h_hbm, rank_hbm, hist, keysb, rankb, shf):
    w = _wid()
    pltpu.sync_copy(keys_hbm.at[pl.ds(w * CHUNK, CHUNK)], keysb)

    zero16 = jnp.zeros((16,), jnp.int32)

    def zbody(j, c):
        hist[pl.ds(j * 16, 16)] = zero16
        return c

    lax.fori_loop(0, NB // 16, zbody, 0)

    neg16 = jnp.full((16,), -1, jnp.int32)
    shf[pl.ds(0, 16)] = neg16
    shf[pl.ds(16, 16)] = neg16
    shf[pl.ds(32, 16)] = neg16

    def body(i, c):
        off = i * 16
        kv = keysb[pl.ds(off, 16)]
        shf[pl.ds(15, 16)] = kv
        within = jnp.zeros((16,), jnp.int32)
        after = jnp.zeros((16,), jnp.int32)
        for k in range(1, 16):
            lv = shf[pl.ds(15 - k, 16)]
            within = within + jnp.where(lv == kv, 1, 0)
            rv = shf[pl.ds(15 + k, 16)]
            after = after + jnp.where(rv == kv, 1, 0)
        rb = plsc.load_gather(hist, [kv])
        rankb[pl.ds(off, 16)] = rb + within
        plsc.store_scatter(hist, [kv], rb + within + 1, mask=after == 0)
        return c

    lax.fori_loop(0, CHUNK // 16, body, 0)

    pltpu.sync_copy(hist, h_hbm.at[pl.ds(w * NB, NB)])
    pltpu.sync_copy(rankb, rank_hbm.at[pl.ds(w * CHUNK, CHUNK)])


@functools.partial(
    pl.kernel,
    mesh=_MESH,
    compiler_params=pltpu.CompilerParams(needs_layout_passes=False),
    out_type=jax.ShapeDtypeStruct((NE,), jnp.int32),
    scratch_types=[
        pltpu.VMEM((NB,), jnp.int32),
        pltpu.VMEM((CHUNK,), jnp.int32),
        pltpu.VMEM((CHUNK,), jnp.int32),
        pltpu.VMEM((CHUNK,), jnp.int32),
    ],
)
def _k2a(t_hbm, keys_hbm, rank_hbm, pos_hbm, tb, kb, rb, pb):
    w = _wid()
    pltpu.sync_copy(t_hbm.at[pl.ds(w * NB, NB)], tb)
    pltpu.sync_copy(keys_hbm.at[pl.ds(w * CHUNK, CHUNK)], kb)
    pltpu.sync_copy(rank_hbm.at[pl.ds(w * CHUNK, CHUNK)], rb)

    def body(i, c):
        off = i * 16
        kv = kb[pl.ds(off, 16)]
        tv = plsc.load_gather(tb, [kv])
        pb[pl.ds(off, 16)] = tv + rb[pl.ds(off, 16)]
        return c

    lax.fori_loop(0, CHUNK // 16, body, 0)
    pltpu.sync_copy(pb, pos_hbm.at[pl.ds(w * CHUNK, CHUNK)])


_NSUB = CHUNK // 128


@functools.partial(
    pl.kernel,
    mesh=_MESH,
    compiler_params=pltpu.CompilerParams(needs_layout_passes=False,
                                         use_tc_tiling_on_sc=False),
    out_type=jax.ShapeDtypeStruct((NE, 16), jnp.float32),
    scratch_types=[
        pltpu.VMEM((CHUNK,), jnp.int32),
        pltpu.VMEM((4 * CHUNK,), jnp.int32),
        pltpu.VMEM((4 * CHUNK,), jnp.int32),
        pltpu.VMEM((_NSUB, 128), jnp.int32),
        pltpu.VMEM((128, 16), jnp.float32),
        pltpu.VMEM((512, 16), jnp.float32),
        pltpu.VMEM((128, 16), jnp.float32),
        pltpu.VMEM((32,), jnp.float32),
        pltpu.SemaphoreType.DMA,
        pltpu.SemaphoreType.DMA,
    ],
)
def _k2b(pcat_hbm, sel_hbm, nei1_hbm, nei2_hbm, nei3_hbm, nei4_hbm, pos_hbm,
         out_hbm, selb, neib, ub, posb, fb, nb, ob, shf, sem_g, sem_s):
    w = _wid()
    dd = w // 8
    j = w % 8
    pltpu.sync_copy(sel_hbm.at[pl.ds(w * CHUNK, CHUNK)], selb)
    pltpu.sync_copy(pos_hbm.at[w], posb)
    for kd, nref in ((0, nei1_hbm), (1, nei2_hbm), (2, nei3_hbm), (3, nei4_hbm)):
        @pl.when(dd == kd)
        def _(nref=nref, kd=kd):
            ln = CHUNK * (kd + 1)
            pltpu.sync_copy(nref.at[pl.ds(j * ln, ln)], neib.at[pl.ds(0, ln)])

    def to_row(n):
        q = jnp.where(n >= S_SPLIT, 1, 0)
        return (n - q * S_SPLIT) * 8 + q * 4 + dd

    def tsel(i, c):
        off = i * 16
        selb[pl.ds(off, 16)] = to_row(selb[pl.ds(off, 16)])
        return c

    lax.fori_loop(0, CHUNK // 16, tsel, 0)

    iota16 = lax.broadcasted_iota(jnp.int32, (16,), 0)

    def tuni(i, c):
        u0 = i * 16
        uv = u0 + iota16
        el = uv >> 2
        kk = uv & 3
        m = el * (dd + 1) + kk
        raw = plsc.load_gather(neib, [m])
        fillv = FILL_BASE + ((w * (4 * CHUNK) + uv) & 255)
        mg = jnp.where(kk <= dd, raw, fillv)
        ub[pl.ds(u0, 16)] = to_row(mg)
        return c

    lax.fori_loop(0, 4 * CHUNK // 16, tuni, 0)

    shf[pl.ds(16, 16)] = jnp.zeros((16,), jnp.float32)

    def sub(s, c):
        cg = pltpu.async_copy(pcat_hbm.at[selb.at[pl.ds(s * 128, 128)]], fb, sem_g)
        cn0 = pltpu.async_copy(pcat_hbm.at[ub.at[pl.ds(s * 512, 128)]],
                               nb.at[pl.ds(0, 128)], sem_g)
        cn1 = pltpu.async_copy(pcat_hbm.at[ub.at[pl.ds(s * 512 + 128, 128)]],
                               nb.at[pl.ds(128, 128)], sem_g)
        cn2 = pltpu.async_copy(pcat_hbm.at[ub.at[pl.ds(s * 512 + 256, 128)]],
                               nb.at[pl.ds(256, 128)], sem_g)
        cn3 = pltpu.async_copy(pcat_hbm.at[ub.at[pl.ds(s * 512 + 384, 128)]],
                               nb.at[pl.ds(384, 128)], sem_g)
        cg.wait()
        cn0.wait()
        cn1.wait()
        cn2.wait()
        cn3.wait()

        def ebody(e, c2):
            base = e * 4
            acc = (nb[base] + nb[base + 1]) + (nb[base + 2] + nb[base + 3])
            shf[pl.ds(0, 16)] = acc
            sh = shf[pl.ds(8, 16)]
            ob[e] = fb[e] + sh
            return c2

        lax.fori_loop(0, 128, ebody, 0)

        cs = pltpu.async_copy(ob, out_hbm.at[posb.at[s]], sem_s)
        cs.wait()
        return c

    lax.fori_loop(0, _NSUB, sub, 0)


def kernel(is_last_layer, x, edge_index, edge_attr, p,
           p_focal_deg1, p_focal_deg2, p_focal_deg3, p_focal_deg4,
           nei_p_deg1, nei_p_deg2, nei_p_deg3, nei_p_deg4,
           nei_edge_attr_deg1, nei_edge_attr_deg2, nei_edge_attr_deg3, nei_edge_attr_deg4,
           selected_index_deg1, selected_index_deg2, selected_index_deg3, selected_index_deg4,
           nei_index_deg1, nei_index_deg2, nei_index_deg3, nei_index_deg4,
           save_score, W1, W2, W3, W4):
    sels = [selected_index_deg1, selected_index_deg2,
            selected_index_deg3, selected_index_deg4]
    neis = [nei_index_deg1, nei_index_deg2, nei_index_deg3, nei_index_deg4]

    hcats = [jnp.concatenate([w.astype(jnp.float32)[:D_FEAT],
                              w.astype(jnp.float32)[D_FEAT:]], axis=1)
             for w in (W1, W2, W3, W4)]
    z128 = jnp.zeros((D_FEAT, 16), jnp.float32)
    cols = [jnp.concatenate([hcats[u % 4], z128] if u < 4 else [z128, hcats[u % 4]],
                            axis=0) for u in range(8)]
    wbig = jnp.concatenate(cols, axis=1)
    pcat_packed = _project(x.astype(jnp.float32), wbig)
    table = pcat_packed.reshape(TROWS, 16)

    pad_k = jnp.full((PAD_BLK - N_FOCAL,), PAD_KEY, jnp.int32)
    keys = jnp.concatenate(
        [jnp.concatenate([s.astype(jnp.int32), pad_k]) for s in sels])

    pad_z = jnp.zeros((PAD_BLK - N_FOCAL,), jnp.int32)
    sel_raw = jnp.concatenate(
        [jnp.concatenate([s.astype(jnp.int32), pad_z]) for s in sels])

    nei_pads = [
        jnp.pad(neis[d - 1].astype(jnp.int32), (0, (PAD_BLK - N_FOCAL) * d))
        for d in range(1, 5)
    ]

    hflat, rank = _k1(keys)
    tgrid = _offsets(hflat.reshape(NW, NB))
    pos = _k2a(tgrid.reshape(NW * NB), keys, rank)
    out_pad = _k2b(table, sel_raw, *nei_pads, pos.reshape(NW, _NSUB, 128))
    return out_pad[:N_NODES, :NK]


# speedup vs baseline: 2.2444x; 1.0816x over previous
import functools

import jax
import jax.numpy as jnp
from jax import lax
from jax.experimental import pallas as pl
from jax.experimental.pallas import tpu as pltpu
from jax.experimental.pallas import tpu_sc as plsc

N_NODES = 100000
N_FOCAL = 25000
D_FEAT = 128
NK = 8

NW = 32
CHUNK = 3200
PAD_BLK = 25600
NE = NW * CHUNK
NB = 100352
PAD_KEY = N_NODES

S_SPLIT = 50176
RBLK = 512
NREAL = S_SPLIT // RBLK
NZERO = 1
NGRID = NREAL + NZERO
PKROWS = NGRID * RBLK
TROWS = PKROWS * 8
FILL_BASE = 2 * S_SPLIT


def _proj_body(xa_ref, xb_ref, w_ref, o_ref):
    i = pl.program_id(0)
    lane = lax.broadcasted_iota(jnp.int32, (1, 128), 1)
    v = lane & 15
    ddv = (lane >> 4) & 3
    scale = jnp.where(v < 8, 1.0, 1.0 / (ddv + 1).astype(jnp.float32))
    wmat = w_ref[...] * scale
    a = jnp.concatenate([xa_ref[...], xb_ref[...]], axis=1)
    res = lax.dot_general(a, wmat, (((1,), (0,)), ((), ())),
                          preferred_element_type=jnp.float32)
    o_ref[...] = jnp.where(i >= NREAL, 0.0, res)


def _project(x, wbig):
    return pl.pallas_call(
        _proj_body,
        grid=(NGRID,),
        in_specs=[
            pl.BlockSpec((RBLK, D_FEAT), lambda i: (jnp.minimum(i, NREAL - 1), 0)),
            pl.BlockSpec((RBLK, D_FEAT),
                         lambda i: (jnp.minimum(NREAL + i, (N_NODES - 1) // RBLK), 0)),
            pl.BlockSpec((2 * D_FEAT, 128), lambda i: (0, 0)),
        ],
        out_specs=pl.BlockSpec((RBLK, 128), lambda i: (i, 0)),
        out_shape=jax.ShapeDtypeStruct((PKROWS, 128), jnp.float32),
    )(x, x, wbig)


def _offsets_body(h_ref, t_ref, carry_ref):
    pid = pl.program_id(0)

    @pl.when(pid == 0)
    def _():
        carry_ref[0] = 0

    blk = h_ref[...]

    def shift_down0(a, s):
        return jnp.pad(a, ((s, 0), (0, 0)))[: a.shape[0], :]

    def shift_down1(a, s):
        return jnp.pad(a, ((0, 0), (s, 0)))[:, : a.shape[1]]

    cum0 = blk
    s = 1
    while s < 32:
        cum0 = cum0 + shift_down0(cum0, s)
        s *= 2
    wexcl = cum0 - blk

    total = jnp.sum(blk, axis=0, keepdims=True)
    cum1 = total
    s = 1
    while s < 2048:
        cum1 = cum1 + shift_down1(cum1, s)
        s *= 2
    carry = carry_ref[0]
    excl_bins = cum1 - total + carry
    t_ref[...] = wexcl + excl_bins
    carry_ref[0] = carry + jnp.sum(total)


def _offsets(hgrid):
    nblk = NB // 2048
    return pl.pallas_call(
        _offsets_body,
        grid=(nblk,),
        in_specs=[pl.BlockSpec((NW, 2048), lambda i: (0, i))],
        out_specs=pl.BlockSpec((NW, 2048), lambda i: (0, i)),
        out_shape=jax.ShapeDtypeStruct((NW, NB), jnp.int32),
        scratch_shapes=[pltpu.SMEM((1,), jnp.int32)],
    )(hgrid)


_MESH = plsc.VectorSubcoreMesh(core_axis_name="c", subcore_axis_name="s")


def _wid():
    return lax.axis_index("s") * 2 + lax.axis_index("c")


@functools.partial(
    pl.kernel,
    mesh=_MESH,
    compiler_params=pltpu.CompilerParams(needs_layout_passes=False),
    out_type=(
        jax.ShapeDtypeStruct((NW * NB,), jnp.int32),
        jax.ShapeDtypeStruct((NE,), jnp.int32),
    ),
    scratch_types=[
        pltpu.VMEM((NB,), jnp.int32),
        pltpu.VMEM((CHUNK,), jnp.int32),
        pltpu.VMEM((CHUNK,), jnp.int32),
        pltpu.VMEM((48,), jnp.int32),
    ],
)
def _k1(keys_hbm, h_hbm, rank_hbm, hist, keysb, rankb, shf):
    w = _wid()
    pltpu.sync_copy(keys_hbm.at[pl.ds(w * CHUNK, CHUNK)], keysb)

    zero16 = jnp.zeros((16,), jnp.int32)

    def zbody(j, c):
        hist[pl.ds(j * 16, 16)] = zero16
        return c

    lax.fori_loop(0, NB // 16, zbody, 0)

    neg16 = jnp.full((16,), -1, jnp.int32)
    shf[pl.ds(0, 16)] = neg16
    shf[pl.ds(16, 16)] = neg16
    shf[pl.ds(32, 16)] = neg16

    def body(i, c):
        off = i * 16
        kv = keysb[pl.ds(off, 16)]
        shf[pl.ds(15, 16)] = kv
        within = jnp.zeros((16,), jnp.int32)
        after = jnp.zeros((16,), jnp.int32)
        for k in range(1, 16):
            lv = shf[pl.ds(15 - k, 16)]
            within = within + jnp.where(lv == kv, 1, 0)
            rv = shf[pl.ds(15 + k, 16)]
            after = after + jnp.where(rv == kv, 1, 0)
        rb = plsc.load_gather(hist, [kv])
        rankb[pl.ds(off, 16)] = rb + within
        plsc.store_scatter(hist, [kv], rb + within + 1, mask=after == 0)
        return c

    lax.fori_loop(0, CHUNK // 16, body, 0)

    pltpu.sync_copy(hist, h_hbm.at[pl.ds(w * NB, NB)])
    pltpu.sync_copy(rankb, rank_hbm.at[pl.ds(w * CHUNK, CHUNK)])


@functools.partial(
    pl.kernel,
    mesh=_MESH,
    compiler_params=pltpu.CompilerParams(needs_layout_passes=False),
    out_type=jax.ShapeDtypeStruct((NE,), jnp.int32),
    scratch_types=[
        pltpu.VMEM((NB,), jnp.int32),
        pltpu.VMEM((CHUNK,), jnp.int32),
        pltpu.VMEM((CHUNK,), jnp.int32),
        pltpu.VMEM((CHUNK,), jnp.int32),
    ],
)
def _k2a(t_hbm, keys_hbm, rank_hbm, pos_hbm, tb, kb, rb, pb):
    w = _wid()
    pltpu.sync_copy(t_hbm.at[pl.ds(w * NB, NB)], tb)
    pltpu.sync_copy(keys_hbm.at[pl.ds(w * CHUNK, CHUNK)], kb)
    pltpu.sync_copy(rank_hbm.at[pl.ds(w * CHUNK, CHUNK)], rb)

    def body(i, c):
        off = i * 16
        kv = kb[pl.ds(off, 16)]
        tv = plsc.load_gather(tb, [kv])
        pb[pl.ds(off, 16)] = tv + rb[pl.ds(off, 16)]
        return c

    lax.fori_loop(0, CHUNK // 16, body, 0)
    pltpu.sync_copy(pb, pos_hbm.at[pl.ds(w * CHUNK, CHUNK)])


_NSUB = CHUNK // 128


@functools.partial(
    pl.kernel,
    mesh=_MESH,
    compiler_params=pltpu.CompilerParams(needs_layout_passes=False,
                                         use_tc_tiling_on_sc=False),
    out_type=jax.ShapeDtypeStruct((NE, 16), jnp.float32),
    scratch_types=[
        pltpu.VMEM((CHUNK,), jnp.int32),
        pltpu.VMEM((4 * CHUNK,), jnp.int32),
        pltpu.VMEM((4 * CHUNK,), jnp.int32),
        pltpu.VMEM((_NSUB, 128), jnp.int32),
        pltpu.VMEM((2, 128, 16), jnp.float32),
        pltpu.VMEM((2, 512, 16), jnp.float32),
        pltpu.VMEM((2, 128, 16), jnp.float32),
        pltpu.VMEM((32,), jnp.float32),
        pltpu.SemaphoreType.DMA,
        pltpu.SemaphoreType.DMA,
        pltpu.SemaphoreType.DMA,
        pltpu.SemaphoreType.DMA,
    ],
)
def _k2b(pcat_hbm, sel_hbm, nei1_hbm, nei2_hbm, nei3_hbm, nei4_hbm, pos_hbm,
         out_hbm, selb, neib, ub, posb, fb, nb, ob, shf,
         gsem0, gsem1, ssem0, ssem1):
    w = _wid()
    dd = w // 8
    j = w % 8
    pltpu.sync_copy(sel_hbm.at[pl.ds(w * CHUNK, CHUNK)], selb)
    pltpu.sync_copy(pos_hbm.at[w], posb)
    for kd, nref in ((0, nei1_hbm), (1, nei2_hbm), (2, nei3_hbm), (3, nei4_hbm)):
        @pl.when(dd == kd)
        def _(nref=nref, kd=kd):
            ln = CHUNK * (kd + 1)
            pltpu.sync_copy(nref.at[pl.ds(j * ln, ln)], neib.at[pl.ds(0, ln)])

    def to_row(n):
        q = jnp.where(n >= S_SPLIT, 1, 0)
        return (n - q * S_SPLIT) * 8 + q * 4 + dd

    def tsel(i, c):
        off = i * 16
        selb[pl.ds(off, 16)] = to_row(selb[pl.ds(off, 16)])
        return c

    lax.fori_loop(0, CHUNK // 16, tsel, 0)

    iota16 = lax.broadcasted_iota(jnp.int32, (16,), 0)

    def tuni(i, c):
        u0 = i * 16
        uv = u0 + iota16
        el = uv >> 2
        kk = uv & 3
        m = el * (dd + 1) + kk
        raw = plsc.load_gather(neib, [m])
        fillv = FILL_BASE + ((w * (4 * CHUNK) + uv) & 255)
        mg = jnp.where(kk <= dd, raw, fillv)
        ub[pl.ds(u0, 16)] = to_row(mg)
        return c

    lax.fori_loop(0, 4 * CHUNK // 16, tuni, 0)

    shf[pl.ds(16, 16)] = jnp.zeros((16,), jnp.float32)

    gsems = (gsem0, gsem1)
    ssems = (ssem0, ssem1)

    def issue(s):
        par = s % 2
        fpar, npar, gs = fb.at[par], nb.at[par], gsems[par]
        descs = [
            pltpu.async_copy(pcat_hbm.at[selb.at[pl.ds(s * 128, 128)]], fpar, gs)
        ]
        for k in range(4):
            descs.append(pltpu.async_copy(
                pcat_hbm.at[ub.at[pl.ds(s * 512 + k * 128, 128)]],
                npar.at[pl.ds(k * 128, 128)], gs))
        return descs

    gds = {0: issue(0)}
    sds = {}
    for s in range(_NSUB):
        par = s % 2
        if s + 1 < _NSUB:
            gds[s + 1] = issue(s + 1)
        for dsc in gds.pop(s):
            dsc.wait()
        if s >= 2:
            sds.pop(s - 2).wait()
        fpar, npar, opar = fb.at[par], nb.at[par], ob.at[par]

        def ebody(e, c2, npar=npar, fpar=fpar, opar=opar):
            base = e * 4
            acc = (npar[base] + npar[base + 1]) + (npar[base + 2] + npar[base + 3])
            shf[pl.ds(0, 16)] = acc
            sh = shf[pl.ds(8, 16)]
            opar[e] = fpar[e] + sh
            return c2

        lax.fori_loop(0, 128, ebody, 0)
        sds[s] = pltpu.async_copy(opar, out_hbm.at[posb.at[s]], ssems[par])
    sds.pop(_NSUB - 2).wait()
    sds.pop(_NSUB - 1).wait()


def kernel(is_last_layer, x, edge_index, edge_attr, p,
           p_focal_deg1, p_focal_deg2, p_focal_deg3, p_focal_deg4,
           nei_p_deg1, nei_p_deg2, nei_p_deg3, nei_p_deg4,
           nei_edge_attr_deg1, nei_edge_attr_deg2, nei_edge_attr_deg3, nei_edge_attr_deg4,
           selected_index_deg1, selected_index_deg2, selected_index_deg3, selected_index_deg4,
           nei_index_deg1, nei_index_deg2, nei_index_deg3, nei_index_deg4,
           save_score, W1, W2, W3, W4):
    sels = [selected_index_deg1, selected_index_deg2,
            selected_index_deg3, selected_index_deg4]
    neis = [nei_index_deg1, nei_index_deg2, nei_index_deg3, nei_index_deg4]

    hcats = [jnp.concatenate([w.astype(jnp.float32)[:D_FEAT],
                              w.astype(jnp.float32)[D_FEAT:]], axis=1)
             for w in (W1, W2, W3, W4)]
    z128 = jnp.zeros((D_FEAT, 16), jnp.float32)
    cols = [jnp.concatenate([hcats[u % 4], z128] if u < 4 else [z128, hcats[u % 4]],
                            axis=0) for u in range(8)]
    wbig = jnp.concatenate(cols, axis=1)
    pcat_packed = _project(x.astype(jnp.float32), wbig)
    table = pcat_packed.reshape(TROWS, 16)

    pad_k = jnp.full((PAD_BLK - N_FOCAL,), PAD_KEY, jnp.int32)
    keys = jnp.concatenate(
        [jnp.concatenate([s.astype(jnp.int32), pad_k]) for s in sels])

    pad_z = jnp.zeros((PAD_BLK - N_FOCAL,), jnp.int32)
    sel_raw = jnp.concatenate(
        [jnp.concatenate([s.astype(jnp.int32), pad_z]) for s in sels])

    nei_pads = [
        jnp.pad(neis[d - 1].astype(jnp.int32), (0, (PAD_BLK - N_FOCAL) * d))
        for d in range(1, 5)
    ]

    hflat, rank = _k1(keys)
    tgrid = _offsets(hflat.reshape(NW, NB))
    pos = _k2a(tgrid.reshape(NW * NB), keys, rank)
    out_pad = _k2b(table, sel_raw, *nei_pads, pos.reshape(NW, _NSUB, 128))
    return out_pad[:N_NODES, :NK]


# speedup vs baseline: 2.2823x; 1.0169x over previous
import functools

import jax
import jax.numpy as jnp
from jax import lax
from jax.experimental import pallas as pl
from jax.experimental.pallas import tpu as pltpu
from jax.experimental.pallas import tpu_sc as plsc

N_NODES = 100000
N_FOCAL = 25000
D_FEAT = 128
NK = 8

NW = 32
CHUNK = 3200
PAD_BLK = 25600
NE = NW * CHUNK
NB = 100352
PAD_KEY = N_NODES

S_SPLIT = 50176
RBLK = 512
NREAL = S_SPLIT // RBLK
NZERO = 1
NGRID = NREAL + NZERO
PKROWS = NGRID * RBLK
TROWS = PKROWS * 8
FILL_BASE = 2 * S_SPLIT


def _proj_body(xa_ref, xb_ref, w_ref, o_ref):
    i = pl.program_id(0)
    lane = lax.broadcasted_iota(jnp.int32, (1, 128), 1)
    v = lane & 15
    ddv = (lane >> 4) & 3
    scale = jnp.where(v < 8, 1.0, 1.0 / (ddv + 1).astype(jnp.float32))
    wmat = w_ref[...] * scale
    a = jnp.concatenate([xa_ref[...], xb_ref[...]], axis=1)
    res = lax.dot_general(a, wmat, (((1,), (0,)), ((), ())),
                          preferred_element_type=jnp.float32)
    o_ref[...] = jnp.where(i >= NREAL, 0.0, res)


def _project(x, wbig):
    return pl.pallas_call(
        _proj_body,
        grid=(NGRID,),
        in_specs=[
            pl.BlockSpec((RBLK, D_FEAT), lambda i: (jnp.minimum(i, NREAL - 1), 0)),
            pl.BlockSpec((RBLK, D_FEAT),
                         lambda i: (jnp.minimum(NREAL + i, (N_NODES - 1) // RBLK), 0)),
            pl.BlockSpec((2 * D_FEAT, 128), lambda i: (0, 0)),
        ],
        out_specs=pl.BlockSpec((RBLK, 128), lambda i: (i, 0)),
        out_shape=jax.ShapeDtypeStruct((PKROWS, 128), jnp.float32),
    )(x, x, wbig)


def _offsets_body(h_ref, t_ref, carry_ref):
    pid = pl.program_id(0)

    @pl.when(pid == 0)
    def _():
        carry_ref[0] = 0

    blk = h_ref[...]

    def shift_down0(a, s):
        return jnp.pad(a, ((s, 0), (0, 0)))[: a.shape[0], :]

    def shift_down1(a, s):
        return jnp.pad(a, ((0, 0), (s, 0)))[:, : a.shape[1]]

    cum0 = blk
    s = 1
    while s < 32:
        cum0 = cum0 + shift_down0(cum0, s)
        s *= 2
    wexcl = cum0 - blk

    total = jnp.sum(blk, axis=0, keepdims=True)
    cum1 = total
    s = 1
    while s < 2048:
        cum1 = cum1 + shift_down1(cum1, s)
        s *= 2
    carry = carry_ref[0]
    excl_bins = cum1 - total + carry
    t_ref[...] = wexcl + excl_bins
    carry_ref[0] = carry + jnp.sum(total)


def _offsets(hgrid):
    nblk = NB // 2048
    return pl.pallas_call(
        _offsets_body,
        grid=(nblk,),
        in_specs=[pl.BlockSpec((NW, 2048), lambda i: (0, i))],
        out_specs=pl.BlockSpec((NW, 2048), lambda i: (0, i)),
        out_shape=jax.ShapeDtypeStruct((NW, NB), jnp.int32),
        scratch_shapes=[pltpu.SMEM((1,), jnp.int32)],
    )(hgrid)


_MESH = plsc.VectorSubcoreMesh(core_axis_name="c", subcore_axis_name="s")


def _wid():
    return lax.axis_index("s") * 2 + lax.axis_index("c")


@functools.partial(
    pl.kernel,
    mesh=_MESH,
    compiler_params=pltpu.CompilerParams(needs_layout_passes=False),
    out_type=(
        jax.ShapeDtypeStruct((NW * NB,), jnp.int32),
        jax.ShapeDtypeStruct((NE,), jnp.int32),
    ),
    scratch_types=[
        pltpu.VMEM((NB,), jnp.int32),
        pltpu.VMEM((CHUNK,), jnp.int32),
        pltpu.VMEM((CHUNK,), jnp.int32),
    ],
)
def _k1(keys_hbm, h_hbm, rank_hbm, hist, keysb, rankb):
    w = _wid()
    pltpu.sync_copy(keys_hbm.at[pl.ds(w * CHUNK, CHUNK)], keysb)

    zero16 = jnp.zeros((16,), jnp.int32)

    def zbody(j, c):
        hist[pl.ds(j * 16, 16)] = zero16
        return c

    lax.fori_loop(0, NB // 16, zbody, 0)

    def body(i, c):
        off = i * 16
        kv = keysb[pl.ds(off, 16)]
        within, last = plsc.scan_count(kv)
        rb = plsc.load_gather(hist, [kv])
        rankb[pl.ds(off, 16)] = rb + within - 1
        plsc.store_scatter(hist, [kv], rb + within, mask=last)
        return c

    lax.fori_loop(0, CHUNK // 16, body, 0)

    pltpu.sync_copy(hist, h_hbm.at[pl.ds(w * NB, NB)])
    pltpu.sync_copy(rankb, rank_hbm.at[pl.ds(w * CHUNK, CHUNK)])


@functools.partial(
    pl.kernel,
    mesh=_MESH,
    compiler_params=pltpu.CompilerParams(needs_layout_passes=False),
    out_type=jax.ShapeDtypeStruct((NE,), jnp.int32),
    scratch_types=[
        pltpu.VMEM((NB,), jnp.int32),
        pltpu.VMEM((CHUNK,), jnp.int32),
        pltpu.VMEM((CHUNK,), jnp.int32),
        pltpu.VMEM((CHUNK,), jnp.int32),
    ],
)
def _k2a(t_hbm, keys_hbm, rank_hbm, pos_hbm, tb, kb, rb, pb):
    w = _wid()
    pltpu.sync_copy(t_hbm.at[pl.ds(w * NB, NB)], tb)
    pltpu.sync_copy(keys_hbm.at[pl.ds(w * CHUNK, CHUNK)], kb)
    pltpu.sync_copy(rank_hbm.at[pl.ds(w * CHUNK, CHUNK)], rb)

    def body(i, c):
        off = i * 16
        kv = kb[pl.ds(off, 16)]
        tv = plsc.load_gather(tb, [kv])
        pb[pl.ds(off, 16)] = tv + rb[pl.ds(off, 16)]
        return c

    lax.fori_loop(0, CHUNK // 16, body, 0)
    pltpu.sync_copy(pb, pos_hbm.at[pl.ds(w * CHUNK, CHUNK)])


_NSUB = CHUNK // 128


@functools.partial(
    pl.kernel,
    mesh=_MESH,
    compiler_params=pltpu.CompilerParams(needs_layout_passes=False,
                                         use_tc_tiling_on_sc=False),
    out_type=jax.ShapeDtypeStruct((NE, 16), jnp.float32),
    scratch_types=[
        pltpu.VMEM((CHUNK,), jnp.int32),
        pltpu.VMEM((4 * CHUNK,), jnp.int32),
        pltpu.VMEM((4 * CHUNK,), jnp.int32),
        pltpu.VMEM((_NSUB, 128), jnp.int32),
        pltpu.VMEM((2, 128, 16), jnp.float32),
        pltpu.VMEM((2, 512, 16), jnp.float32),
        pltpu.VMEM((2, 128, 16), jnp.float32),
        pltpu.VMEM((32,), jnp.float32),
        pltpu.SemaphoreType.DMA,
        pltpu.SemaphoreType.DMA,
        pltpu.SemaphoreType.DMA,
        pltpu.SemaphoreType.DMA,
    ],
)
def _k2b(pcat_hbm, sel_hbm, nei1_hbm, nei2_hbm, nei3_hbm, nei4_hbm, pos_hbm,
         out_hbm, selb, neib, ub, posb, fb, nb, ob, shf,
         gsem0, gsem1, ssem0, ssem1):
    w = _wid()
    dd = w // 8
    j = w % 8
    pltpu.sync_copy(sel_hbm.at[pl.ds(w * CHUNK, CHUNK)], selb)
    pltpu.sync_copy(pos_hbm.at[w], posb)
    for kd, nref in ((0, nei1_hbm), (1, nei2_hbm), (2, nei3_hbm), (3, nei4_hbm)):
        @pl.when(dd == kd)
        def _(nref=nref, kd=kd):
            ln = CHUNK * (kd + 1)
            pltpu.sync_copy(nref.at[pl.ds(j * ln, ln)], neib.at[pl.ds(0, ln)])

    def to_row(n):
        q = jnp.where(n >= S_SPLIT, 1, 0)
        return (n - q * S_SPLIT) * 8 + q * 4 + dd

    def tsel(i, c):
        off = i * 16
        selb[pl.ds(off, 16)] = to_row(selb[pl.ds(off, 16)])
        return c

    lax.fori_loop(0, CHUNK // 16, tsel, 0)

    iota16 = lax.broadcasted_iota(jnp.int32, (16,), 0)

    def tuni(i, c):
        u0 = i * 16
        uv = u0 + iota16
        el = uv >> 2
        kk = uv & 3
        m = el * (dd + 1) + kk
        raw = plsc.load_gather(neib, [m])
        fillv = FILL_BASE + ((w * (4 * CHUNK) + uv) & 255)
        mg = jnp.where(kk <= dd, raw, fillv)
        ub[pl.ds(u0, 16)] = to_row(mg)
        return c

    lax.fori_loop(0, 4 * CHUNK // 16, tuni, 0)

    shf[pl.ds(16, 16)] = jnp.zeros((16,), jnp.float32)

    gsems = (gsem0, gsem1)
    ssems = (ssem0, ssem1)

    def issue(s):
        par = s % 2
        fpar, npar, gs = fb.at[par], nb.at[par], gsems[par]
        descs = [
            pltpu.async_copy(pcat_hbm.at[selb.at[pl.ds(s * 128, 128)]], fpar, gs)
        ]
        for k in range(4):
            descs.append(pltpu.async_copy(
                pcat_hbm.at[ub.at[pl.ds(s * 512 + k * 128, 128)]],
                npar.at[pl.ds(k * 128, 128)], gs))
        return descs

    gds = {0: issue(0)}
    sds = {}
    for s in range(_NSUB):
        par = s % 2
        if s + 1 < _NSUB:
            gds[s + 1] = issue(s + 1)
        for dsc in gds.pop(s):
            dsc.wait()
        if s >= 2:
            sds.pop(s - 2).wait()
        fpar, npar, opar = fb.at[par], nb.at[par], ob.at[par]

        def ebody(e, c2, npar=npar, fpar=fpar, opar=opar):
            base = e * 4
            acc = (npar[base] + npar[base + 1]) + (npar[base + 2] + npar[base + 3])
            shf[pl.ds(0, 16)] = acc
            sh = shf[pl.ds(8, 16)]
            opar[e] = fpar[e] + sh
            return c2

        lax.fori_loop(0, 128, ebody, 0)
        sds[s] = pltpu.async_copy(opar, out_hbm.at[posb.at[s]], ssems[par])
    sds.pop(_NSUB - 2).wait()
    sds.pop(_NSUB - 1).wait()


def kernel(is_last_layer, x, edge_index, edge_attr, p,
           p_focal_deg1, p_focal_deg2, p_focal_deg3, p_focal_deg4,
           nei_p_deg1, nei_p_deg2, nei_p_deg3, nei_p_deg4,
           nei_edge_attr_deg1, nei_edge_attr_deg2, nei_edge_attr_deg3, nei_edge_attr_deg4,
           selected_index_deg1, selected_index_deg2, selected_index_deg3, selected_index_deg4,
           nei_index_deg1, nei_index_deg2, nei_index_deg3, nei_index_deg4,
           save_score, W1, W2, W3, W4):
    sels = [selected_index_deg1, selected_index_deg2,
            selected_index_deg3, selected_index_deg4]
    neis = [nei_index_deg1, nei_index_deg2, nei_index_deg3, nei_index_deg4]

    hcats = [jnp.concatenate([w.astype(jnp.float32)[:D_FEAT],
                              w.astype(jnp.float32)[D_FEAT:]], axis=1)
             for w in (W1, W2, W3, W4)]
    z128 = jnp.zeros((D_FEAT, 16), jnp.float32)
    cols = [jnp.concatenate([hcats[u % 4], z128] if u < 4 else [z128, hcats[u % 4]],
                            axis=0) for u in range(8)]
    wbig = jnp.concatenate(cols, axis=1)
    pcat_packed = _project(x.astype(jnp.float32), wbig)
    table = pcat_packed.reshape(TROWS, 16)

    pad_k = jnp.full((PAD_BLK - N_FOCAL,), PAD_KEY, jnp.int32)
    keys = jnp.concatenate(
        [jnp.concatenate([s.astype(jnp.int32), pad_k]) for s in sels])

    pad_z = jnp.zeros((PAD_BLK - N_FOCAL,), jnp.int32)
    sel_raw = jnp.concatenate(
        [jnp.concatenate([s.astype(jnp.int32), pad_z]) for s in sels])

    nei_pads = [
        jnp.pad(neis[d - 1].astype(jnp.int32), (0, (PAD_BLK - N_FOCAL) * d))
        for d in range(1, 5)
    ]

    hflat, rank = _k1(keys)
    tgrid = _offsets(hflat.reshape(NW, NB))
    pos = _k2a(tgrid.reshape(NW * NB), keys, rank)
    out_pad = _k2b(table, sel_raw, *nei_pads, pos.reshape(NW, _NSUB, 128))
    return out_pad[:N_NODES, :NK]


# speedup vs baseline: 2.6101x; 1.1436x over previous
import functools

import jax
import jax.numpy as jnp
from jax import lax
from jax.experimental import pallas as pl
from jax.experimental.pallas import tpu as pltpu
from jax.experimental.pallas import tpu_sc as plsc

N_NODES = 100000
N_FOCAL = 25000
D_FEAT = 128
NK = 8

NW = 32
CHUNK = 3200
PAD_BLK = 25600
NE = NW * CHUNK
NB = 100352
PAD_KEY = N_NODES

S_SPLIT = 50176
RBLK = 512
NREAL = S_SPLIT // RBLK
NZERO = 1
NGRID = NREAL + NZERO
PKROWS = NGRID * RBLK
TROWS = PKROWS * 8
FILL_BASE = 2 * S_SPLIT


def _proj_body(xa_ref, xb_ref, w_ref, o_ref):
    i = pl.program_id(0)
    lane = lax.broadcasted_iota(jnp.int32, (1, 128), 1)
    v = lane & 15
    ddv = (lane >> 4) & 3
    scale = jnp.where(v < 8, 1.0, 1.0 / (ddv + 1).astype(jnp.float32))
    wmat = w_ref[...] * scale
    a = jnp.concatenate([xa_ref[...], xb_ref[...]], axis=1)
    res = lax.dot_general(a, wmat, (((1,), (0,)), ((), ())),
                          preferred_element_type=jnp.float32)
    o_ref[...] = jnp.where(i >= NREAL, 0.0, res)


def _project(x, wbig):
    return pl.pallas_call(
        _proj_body,
        grid=(NGRID,),
        in_specs=[
            pl.BlockSpec((RBLK, D_FEAT), lambda i: (jnp.minimum(i, NREAL - 1), 0)),
            pl.BlockSpec((RBLK, D_FEAT),
                         lambda i: (jnp.minimum(NREAL + i, (N_NODES - 1) // RBLK), 0)),
            pl.BlockSpec((2 * D_FEAT, 128), lambda i: (0, 0)),
        ],
        out_specs=pl.BlockSpec((RBLK, 128), lambda i: (i, 0)),
        out_shape=jax.ShapeDtypeStruct((PKROWS, 128), jnp.float32),
    )(x, x, wbig)


def _offsets_body(h_ref, t_ref, carry_ref):
    pid = pl.program_id(0)

    @pl.when(pid == 0)
    def _():
        carry_ref[0] = 0

    blk = h_ref[...]

    def shift_down0(a, s):
        return jnp.pad(a, ((s, 0), (0, 0)))[: a.shape[0], :]

    def shift_down1(a, s):
        return jnp.pad(a, ((0, 0), (s, 0)))[:, : a.shape[1]]

    cum0 = blk
    s = 1
    while s < 32:
        cum0 = cum0 + shift_down0(cum0, s)
        s *= 2
    wexcl = cum0 - blk

    total = jnp.sum(blk, axis=0, keepdims=True)
    cum1 = total
    s = 1
    while s < total.shape[1]:
        cum1 = cum1 + shift_down1(cum1, s)
        s *= 2
    carry = carry_ref[0]
    excl_bins = cum1 - total + carry
    t_ref[...] = wexcl + excl_bins
    carry_ref[0] = carry + jnp.sum(total)


def _offsets(hgrid):
    nblk = NB // 14336
    return pl.pallas_call(
        _offsets_body,
        grid=(nblk,),
        in_specs=[pl.BlockSpec((NW, 14336), lambda i: (0, i))],
        out_specs=pl.BlockSpec((NW, 14336), lambda i: (0, i)),
        out_shape=jax.ShapeDtypeStruct((NW, NB), jnp.int32),
        scratch_shapes=[pltpu.SMEM((1,), jnp.int32)],
    )(hgrid)


_MESH = plsc.VectorSubcoreMesh(core_axis_name="c", subcore_axis_name="s")


def _wid():
    return lax.axis_index("s") * 2 + lax.axis_index("c")


@functools.partial(
    pl.kernel,
    mesh=_MESH,
    compiler_params=pltpu.CompilerParams(needs_layout_passes=False),
    out_type=(
        jax.ShapeDtypeStruct((NW * NB,), jnp.int32),
        jax.ShapeDtypeStruct((NE,), jnp.int32),
    ),
    scratch_types=[
        pltpu.VMEM((NB,), jnp.int32),
        pltpu.VMEM((CHUNK,), jnp.int32),
        pltpu.VMEM((CHUNK,), jnp.int32),
    ],
)
def _k1(keys_hbm, h_hbm, rank_hbm, hist, keysb, rankb):
    w = _wid()
    pltpu.sync_copy(keys_hbm.at[pl.ds(w * CHUNK, CHUNK)], keysb)

    zero16 = jnp.zeros((16,), jnp.int32)

    def zbody(j, c):
        for t in range(8):
            hist[pl.ds(j * 128 + t * 16, 16)] = zero16
        return c

    lax.fori_loop(0, NB // 128, zbody, 0)

    def body(i, c):
        off = i * 16
        kv = keysb[pl.ds(off, 16)]
        within, last = plsc.scan_count(kv)
        rb = plsc.load_gather(hist, [kv])
        rankb[pl.ds(off, 16)] = rb + within - 1
        plsc.store_scatter(hist, [kv], rb + within, mask=last)
        return c

    lax.fori_loop(0, CHUNK // 16, body, 0)

    pltpu.sync_copy(hist, h_hbm.at[pl.ds(w * NB, NB)])
    pltpu.sync_copy(rankb, rank_hbm.at[pl.ds(w * CHUNK, CHUNK)])


@functools.partial(
    pl.kernel,
    mesh=_MESH,
    compiler_params=pltpu.CompilerParams(needs_layout_passes=False),
    out_type=jax.ShapeDtypeStruct((NE,), jnp.int32),
    scratch_types=[
        pltpu.VMEM((NB,), jnp.int32),
        pltpu.VMEM((CHUNK,), jnp.int32),
        pltpu.VMEM((CHUNK,), jnp.int32),
        pltpu.VMEM((CHUNK,), jnp.int32),
    ],
)
def _k2a(t_hbm, keys_hbm, rank_hbm, pos_hbm, tb, kb, rb, pb):
    w = _wid()
    pltpu.sync_copy(t_hbm.at[pl.ds(w * NB, NB)], tb)
    pltpu.sync_copy(keys_hbm.at[pl.ds(w * CHUNK, CHUNK)], kb)
    pltpu.sync_copy(rank_hbm.at[pl.ds(w * CHUNK, CHUNK)], rb)

    def body(i, c):
        off = i * 16
        kv = kb[pl.ds(off, 16)]
        tv = plsc.load_gather(tb, [kv])
        pb[pl.ds(off, 16)] = tv + rb[pl.ds(off, 16)]
        return c

    lax.fori_loop(0, CHUNK // 16, body, 0)
    pltpu.sync_copy(pb, pos_hbm.at[pl.ds(w * CHUNK, CHUNK)])


_NSUB = CHUNK // 128


@functools.partial(
    pl.kernel,
    mesh=_MESH,
    compiler_params=pltpu.CompilerParams(needs_layout_passes=False,
                                         use_tc_tiling_on_sc=False),
    out_type=jax.ShapeDtypeStruct((NE, 16), jnp.float32),
    scratch_types=[
        pltpu.VMEM((CHUNK,), jnp.int32),
        pltpu.VMEM((4 * CHUNK,), jnp.int32),
        pltpu.VMEM((4 * CHUNK,), jnp.int32),
        pltpu.VMEM((_NSUB, 128), jnp.int32),
        pltpu.VMEM((2, 128, 16), jnp.float32),
        pltpu.VMEM((2, 512, 16), jnp.float32),
        pltpu.VMEM((2, 128, 16), jnp.float32),
        pltpu.VMEM((128,), jnp.float32),
        pltpu.SemaphoreType.DMA,
        pltpu.SemaphoreType.DMA,
        pltpu.SemaphoreType.DMA,
        pltpu.SemaphoreType.DMA,
    ],
)
def _k2b(pcat_hbm, sel_hbm, nei1_hbm, nei2_hbm, nei3_hbm, nei4_hbm, pos_hbm,
         out_hbm, selb, neib, ub, posb, fb, nb, ob, shf,
         gsem0, gsem1, ssem0, ssem1):
    w = _wid()
    dd = w // 8
    j = w % 8
    pltpu.sync_copy(sel_hbm.at[pl.ds(w * CHUNK, CHUNK)], selb)
    pltpu.sync_copy(pos_hbm.at[w], posb)
    for kd, nref in ((0, nei1_hbm), (1, nei2_hbm), (2, nei3_hbm), (3, nei4_hbm)):
        @pl.when(dd == kd)
        def _(nref=nref, kd=kd):
            ln = CHUNK * (kd + 1)
            pltpu.sync_copy(nref.at[pl.ds(j * ln, ln)], neib.at[pl.ds(0, ln)])

    def to_row(n):
        q = jnp.where(n >= S_SPLIT, 1, 0)
        return (n - q * S_SPLIT) * 8 + q * 4 + dd

    def tsel(i, c):
        off = i * 16
        selb[pl.ds(off, 16)] = to_row(selb[pl.ds(off, 16)])
        return c

    lax.fori_loop(0, CHUNK // 16, tsel, 0)

    iota16 = lax.broadcasted_iota(jnp.int32, (16,), 0)

    def tuni(i, c):
        u0 = i * 16
        uv = u0 + iota16
        el = uv >> 2
        kk = uv & 3
        m = el * (dd + 1) + kk
        raw = plsc.load_gather(neib, [m])
        fillv = FILL_BASE + ((w * (4 * CHUNK) + uv) & 255)
        mg = jnp.where(kk <= dd, raw, fillv)
        ub[pl.ds(u0, 16)] = to_row(mg)
        return c

    lax.fori_loop(0, 4 * CHUNK // 16, tuni, 0)

    for t in range(4):
        shf[pl.ds(t * 32 + 16, 16)] = jnp.zeros((16,), jnp.float32)

    gsems = (gsem0, gsem1)
    ssems = (ssem0, ssem1)

    def issue(s):
        par = s % 2
        fpar, npar, gs = fb.at[par], nb.at[par], gsems[par]
        descs = [
            pltpu.async_copy(pcat_hbm.at[selb.at[pl.ds(s * 128, 128)]], fpar, gs)
        ]
        for k in range(4):
            descs.append(pltpu.async_copy(
                pcat_hbm.at[ub.at[pl.ds(s * 512 + k * 128, 128)]],
                npar.at[pl.ds(k * 128, 128)], gs))
        return descs

    gds = {0: issue(0)}
    sds = {}
    for s in range(_NSUB):
        par = s % 2
        if s + 1 < _NSUB:
            gds[s + 1] = issue(s + 1)
        for dsc in gds.pop(s):
            dsc.wait()
        if s >= 2:
            sds.pop(s - 2).wait()
        fpar, npar, opar = fb.at[par], nb.at[par], ob.at[par]

        def ebody(i, c2, npar=npar, fpar=fpar, opar=opar):
            accs = []
            for t in range(4):
                e = i * 4 + t
                base = e * 4
                acc = (npar[base] + npar[base + 1]) + (npar[base + 2] + npar[base + 3])
                shf[pl.ds(t * 32, 16)] = acc
                accs.append(e)
            for t, e in enumerate(accs):
                sh = shf[pl.ds(t * 32 + 8, 16)]
                opar[e] = fpar[e] + sh
            return c2

        lax.fori_loop(0, 32, ebody, 0)
        sds[s] = pltpu.async_copy(opar, out_hbm.at[posb.at[s]], ssems[par])
    sds.pop(_NSUB - 2).wait()
    sds.pop(_NSUB - 1).wait()


def kernel(is_last_layer, x, edge_index, edge_attr, p,
           p_focal_deg1, p_focal_deg2, p_focal_deg3, p_focal_deg4,
           nei_p_deg1, nei_p_deg2, nei_p_deg3, nei_p_deg4,
           nei_edge_attr_deg1, nei_edge_attr_deg2, nei_edge_attr_deg3, nei_edge_attr_deg4,
           selected_index_deg1, selected_index_deg2, selected_index_deg3, selected_index_deg4,
           nei_index_deg1, nei_index_deg2, nei_index_deg3, nei_index_deg4,
           save_score, W1, W2, W3, W4):
    sels = [selected_index_deg1, selected_index_deg2,
            selected_index_deg3, selected_index_deg4]
    neis = [nei_index_deg1, nei_index_deg2, nei_index_deg3, nei_index_deg4]

    hcats = [jnp.concatenate([w.astype(jnp.float32)[:D_FEAT],
                              w.astype(jnp.float32)[D_FEAT:]], axis=1)
             for w in (W1, W2, W3, W4)]
    z128 = jnp.zeros((D_FEAT, 16), jnp.float32)
    cols = [jnp.concatenate([hcats[u % 4], z128] if u < 4 else [z128, hcats[u % 4]],
                            axis=0) for u in range(8)]
    wbig = jnp.concatenate(cols, axis=1)
    pcat_packed = _project(x.astype(jnp.float32), wbig)
    table = pcat_packed.reshape(TROWS, 16)

    pad_k = jnp.full((PAD_BLK - N_FOCAL,), PAD_KEY, jnp.int32)
    keys = jnp.concatenate(
        [jnp.concatenate([s.astype(jnp.int32), pad_k]) for s in sels])

    pad_z = jnp.zeros((PAD_BLK - N_FOCAL,), jnp.int32)
    sel_raw = jnp.concatenate(
        [jnp.concatenate([s.astype(jnp.int32), pad_z]) for s in sels])

    nei_pads = [
        jnp.pad(neis[d - 1].astype(jnp.int32), (0, (PAD_BLK - N_FOCAL) * d))
        for d in range(1, 5)
    ]

    hflat, rank = _k1(keys)
    tgrid = _offsets(hflat.reshape(NW, NB))
    pos = _k2a(tgrid.reshape(NW * NB), keys, rank)
    out_pad = _k2b(table, sel_raw, *nei_pads, pos.reshape(NW, _NSUB, 128))
    return out_pad[:N_NODES, :NK]


# speedup vs baseline: 2.6198x; 1.0037x over previous
import functools

import jax
import jax.numpy as jnp
from jax import lax
from jax.experimental import pallas as pl
from jax.experimental.pallas import tpu as pltpu
from jax.experimental.pallas import tpu_sc as plsc

N_NODES = 100000
N_FOCAL = 25000
D_FEAT = 128
NK = 8

NW = 32
CHUNK = 3200
PAD_BLK = 25600
NE = NW * CHUNK
NB = 100352
PAD_KEY = N_NODES

S_SPLIT = 50176
RBLK = 512
NREAL = S_SPLIT // RBLK
NZERO = 1
NGRID = NREAL + NZERO
PKROWS = NGRID * RBLK
TROWS = PKROWS * 8
FILL_BASE = 2 * S_SPLIT


def _proj_body(xa_ref, xb_ref, w_ref, o_ref):
    i = pl.program_id(0)
    lane = lax.broadcasted_iota(jnp.int32, (1, 128), 1)
    v = lane & 15
    ddv = (lane >> 4) & 3
    scale = jnp.where(v < 8, 1.0, 1.0 / (ddv + 1).astype(jnp.float32))
    wmat = w_ref[...] * scale
    a = jnp.concatenate([xa_ref[...], xb_ref[...]], axis=1)
    res = lax.dot_general(a, wmat, (((1,), (0,)), ((), ())),
                          preferred_element_type=jnp.float32)
    o_ref[...] = jnp.where(i >= NREAL, 0.0, res)


def _project(x, wbig):
    return pl.pallas_call(
        _proj_body,
        grid=(NGRID,),
        in_specs=[
            pl.BlockSpec((RBLK, D_FEAT), lambda i: (jnp.minimum(i, NREAL - 1), 0)),
            pl.BlockSpec((RBLK, D_FEAT),
                         lambda i: (jnp.minimum(NREAL + i, (N_NODES - 1) // RBLK), 0)),
            pl.BlockSpec((2 * D_FEAT, 128), lambda i: (0, 0)),
        ],
        out_specs=pl.BlockSpec((RBLK, 128), lambda i: (i, 0)),
        out_shape=jax.ShapeDtypeStruct((PKROWS, 128), jnp.float32),
    )(x, x, wbig)


def _offsets_body(h_ref, t_ref, carry_ref):
    pid = pl.program_id(0)

    @pl.when(pid == 0)
    def _():
        carry_ref[0] = 0

    blk = h_ref[...]

    def shift_down0(a, s):
        return jnp.pad(a, ((s, 0), (0, 0)))[: a.shape[0], :]

    def shift_down1(a, s):
        return jnp.pad(a, ((0, 0), (s, 0)))[:, : a.shape[1]]

    cum0 = blk
    s = 1
    while s < 32:
        cum0 = cum0 + shift_down0(cum0, s)
        s *= 2
    wexcl = cum0 - blk

    total = jnp.sum(blk, axis=0, keepdims=True)
    cum1 = total
    s = 1
    while s < total.shape[1]:
        cum1 = cum1 + shift_down1(cum1, s)
        s *= 2
    carry = carry_ref[0]
    excl_bins = cum1 - total + carry
    t_ref[...] = wexcl + excl_bins
    carry_ref[0] = carry + jnp.sum(total)


def _offsets(hgrid):
    nblk = NB // 14336
    return pl.pallas_call(
        _offsets_body,
        grid=(nblk,),
        in_specs=[pl.BlockSpec((NW, 14336), lambda i: (0, i))],
        out_specs=pl.BlockSpec((NW, 14336), lambda i: (0, i)),
        out_shape=jax.ShapeDtypeStruct((NW, NB), jnp.int32),
        scratch_shapes=[pltpu.SMEM((1,), jnp.int32)],
    )(hgrid)


_MESH = plsc.VectorSubcoreMesh(core_axis_name="c", subcore_axis_name="s")


def _wid():
    return lax.axis_index("s") * 2 + lax.axis_index("c")


@functools.partial(
    pl.kernel,
    mesh=_MESH,
    compiler_params=pltpu.CompilerParams(needs_layout_passes=False),
    out_type=(
        jax.ShapeDtypeStruct((NW * NB,), jnp.int32),
        jax.ShapeDtypeStruct((NE,), jnp.int32),
    ),
    scratch_types=[
        pltpu.VMEM((NB,), jnp.int32),
        pltpu.VMEM((CHUNK,), jnp.int32),
        pltpu.VMEM((CHUNK,), jnp.int32),
    ],
)
def _k1(keys_hbm, h_hbm, rank_hbm, hist, keysb, rankb):
    w = _wid()
    pltpu.sync_copy(keys_hbm.at[pl.ds(w * CHUNK, CHUNK)], keysb)

    zero16 = jnp.zeros((16,), jnp.int32)

    def zbody(j, c):
        for t in range(8):
            hist[pl.ds(j * 128 + t * 16, 16)] = zero16
        return c

    lax.fori_loop(0, NB // 128, zbody, 0)

    def body(i, c):
        off = i * 16
        kv = keysb[pl.ds(off, 16)]
        within, last = plsc.scan_count(kv)
        rb = plsc.load_gather(hist, [kv])
        rankb[pl.ds(off, 16)] = rb + within - 1
        plsc.store_scatter(hist, [kv], rb + within, mask=last)
        return c

    lax.fori_loop(0, CHUNK // 16, body, 0)

    pltpu.sync_copy(hist, h_hbm.at[pl.ds(w * NB, NB)])
    pltpu.sync_copy(rankb, rank_hbm.at[pl.ds(w * CHUNK, CHUNK)])


@functools.partial(
    pl.kernel,
    mesh=_MESH,
    compiler_params=pltpu.CompilerParams(needs_layout_passes=False),
    out_type=jax.ShapeDtypeStruct((NE,), jnp.int32),
    scratch_types=[
        pltpu.VMEM((NB,), jnp.int32),
        pltpu.VMEM((CHUNK,), jnp.int32),
        pltpu.VMEM((CHUNK,), jnp.int32),
        pltpu.VMEM((CHUNK,), jnp.int32),
    ],
)
def _k2a(t_hbm, keys_hbm, rank_hbm, pos_hbm, tb, kb, rb, pb):
    w = _wid()
    pltpu.sync_copy(t_hbm.at[pl.ds(w * NB, NB)], tb)
    pltpu.sync_copy(keys_hbm.at[pl.ds(w * CHUNK, CHUNK)], kb)
    pltpu.sync_copy(rank_hbm.at[pl.ds(w * CHUNK, CHUNK)], rb)

    def body(i, c):
        off = i * 16
        kv = kb[pl.ds(off, 16)]
        tv = plsc.load_gather(tb, [kv])
        pb[pl.ds(off, 16)] = tv + rb[pl.ds(off, 16)]
        return c

    lax.fori_loop(0, CHUNK // 16, body, 0)
    pltpu.sync_copy(pb, pos_hbm.at[pl.ds(w * CHUNK, CHUNK)])


_NSUB = CHUNK // 128


@functools.partial(
    pl.kernel,
    mesh=_MESH,
    compiler_params=pltpu.CompilerParams(needs_layout_passes=False,
                                         use_tc_tiling_on_sc=False),
    out_type=jax.ShapeDtypeStruct((NE, 16), jnp.float32),
    scratch_types=[
        pltpu.VMEM((CHUNK,), jnp.int32),
        pltpu.VMEM((4 * CHUNK,), jnp.int32),
        pltpu.VMEM((4 * CHUNK,), jnp.int32),
        pltpu.VMEM((_NSUB, 128), jnp.int32),
        pltpu.VMEM((4, 128, 16), jnp.float32),
        pltpu.VMEM((4, 512, 16), jnp.float32),
        pltpu.VMEM((4, 128, 16), jnp.float32),
        pltpu.VMEM((128,), jnp.float32),
        pltpu.SemaphoreType.DMA,
        pltpu.SemaphoreType.DMA,
        pltpu.SemaphoreType.DMA,
        pltpu.SemaphoreType.DMA,
        pltpu.SemaphoreType.DMA,
        pltpu.SemaphoreType.DMA,
        pltpu.SemaphoreType.DMA,
        pltpu.SemaphoreType.DMA,
    ],
)
def _k2b(pcat_hbm, sel_hbm, nei1_hbm, nei2_hbm, nei3_hbm, nei4_hbm, pos_hbm,
         out_hbm, selb, neib, ub, posb, fb, nb, ob, shf,
         gsem0, gsem1, gsem2, gsem3, ssem0, ssem1, ssem2, ssem3):
    w = _wid()
    dd = w // 8
    j = w % 8
    pltpu.sync_copy(sel_hbm.at[pl.ds(w * CHUNK, CHUNK)], selb)
    pltpu.sync_copy(pos_hbm.at[w], posb)
    for kd, nref in ((0, nei1_hbm), (1, nei2_hbm), (2, nei3_hbm), (3, nei4_hbm)):
        @pl.when(dd == kd)
        def _(nref=nref, kd=kd):
            ln = CHUNK * (kd + 1)
            pltpu.sync_copy(nref.at[pl.ds(j * ln, ln)], neib.at[pl.ds(0, ln)])

    def to_row(n):
        q = jnp.where(n >= S_SPLIT, 1, 0)
        return (n - q * S_SPLIT) * 8 + q * 4 + dd

    def tsel(i, c):
        off = i * 16
        selb[pl.ds(off, 16)] = to_row(selb[pl.ds(off, 16)])
        return c

    lax.fori_loop(0, CHUNK // 16, tsel, 0)

    iota16 = lax.broadcasted_iota(jnp.int32, (16,), 0)

    def tuni(i, c):
        u0 = i * 16
        uv = u0 + iota16
        el = uv >> 2
        kk = uv & 3
        m = el * (dd + 1) + kk
        raw = plsc.load_gather(neib, [m])
        fillv = FILL_BASE + ((w * (4 * CHUNK) + uv) & 255)
        mg = jnp.where(kk <= dd, raw, fillv)
        ub[pl.ds(u0, 16)] = to_row(mg)
        return c

    lax.fori_loop(0, 4 * CHUNK // 16, tuni, 0)

    for t in range(4):
        shf[pl.ds(t * 32 + 16, 16)] = jnp.zeros((16,), jnp.float32)

    gsems = (gsem0, gsem1, gsem2, gsem3)
    ssems = (ssem0, ssem1, ssem2, ssem3)

    def issue(s):
        par = s % 4
        fpar, npar, gs = fb.at[par], nb.at[par], gsems[par]
        descs = [
            pltpu.async_copy(pcat_hbm.at[selb.at[pl.ds(s * 128, 128)]], fpar, gs)
        ]
        for k in range(4):
            descs.append(pltpu.async_copy(
                pcat_hbm.at[ub.at[pl.ds(s * 512 + k * 128, 128)]],
                npar.at[pl.ds(k * 128, 128)], gs))
        return descs

    gds = {0: issue(0), 1: issue(1), 2: issue(2)}
    sds = {}
    for s in range(_NSUB):
        par = s % 4
        if s + 3 < _NSUB:
            gds[s + 3] = issue(s + 3)
        for dsc in gds.pop(s):
            dsc.wait()
        if s >= 4:
            sds.pop(s - 4).wait()
        fpar, npar, opar = fb.at[par], nb.at[par], ob.at[par]

        def ebody(i, c2, npar=npar, fpar=fpar, opar=opar):
            accs = []
            for t in range(4):
                e = i * 4 + t
                base = e * 4
                acc = (npar[base] + npar[base + 1]) + (npar[base + 2] + npar[base + 3])
                shf[pl.ds(t * 32, 16)] = acc
                accs.append(e)
            for t, e in enumerate(accs):
                sh = shf[pl.ds(t * 32 + 8, 16)]
                opar[e] = fpar[e] + sh
            return c2

        lax.fori_loop(0, 32, ebody, 0)
        sds[s] = pltpu.async_copy(opar, out_hbm.at[posb.at[s]], ssems[par])
    for s in sorted(sds):
        sds.pop(s).wait()


def kernel(is_last_layer, x, edge_index, edge_attr, p,
           p_focal_deg1, p_focal_deg2, p_focal_deg3, p_focal_deg4,
           nei_p_deg1, nei_p_deg2, nei_p_deg3, nei_p_deg4,
           nei_edge_attr_deg1, nei_edge_attr_deg2, nei_edge_attr_deg3, nei_edge_attr_deg4,
           selected_index_deg1, selected_index_deg2, selected_index_deg3, selected_index_deg4,
           nei_index_deg1, nei_index_deg2, nei_index_deg3, nei_index_deg4,
           save_score, W1, W2, W3, W4):
    sels = [selected_index_deg1, selected_index_deg2,
            selected_index_deg3, selected_index_deg4]
    neis = [nei_index_deg1, nei_index_deg2, nei_index_deg3, nei_index_deg4]

    hcats = [jnp.concatenate([w.astype(jnp.float32)[:D_FEAT],
                              w.astype(jnp.float32)[D_FEAT:]], axis=1)
             for w in (W1, W2, W3, W4)]
    z128 = jnp.zeros((D_FEAT, 16), jnp.float32)
    cols = [jnp.concatenate([hcats[u % 4], z128] if u < 4 else [z128, hcats[u % 4]],
                            axis=0) for u in range(8)]
    wbig = jnp.concatenate(cols, axis=1)
    pcat_packed = _project(x.astype(jnp.float32), wbig)
    table = pcat_packed.reshape(TROWS, 16)

    pad_k = jnp.full((PAD_BLK - N_FOCAL,), PAD_KEY, jnp.int32)
    keys = jnp.concatenate(
        [jnp.concatenate([s.astype(jnp.int32), pad_k]) for s in sels])

    pad_z = jnp.zeros((PAD_BLK - N_FOCAL,), jnp.int32)
    sel_raw = jnp.concatenate(
        [jnp.concatenate([s.astype(jnp.int32), pad_z]) for s in sels])

    nei_pads = [
        jnp.pad(neis[d - 1].astype(jnp.int32), (0, (PAD_BLK - N_FOCAL) * d))
        for d in range(1, 5)
    ]

    hflat, rank = _k1(keys)
    tgrid = _offsets(hflat.reshape(NW, NB))
    pos = _k2a(tgrid.reshape(NW * NB), keys, rank)
    out_pad = _k2b(table, sel_raw, *nei_pads, pos.reshape(NW, _NSUB, 128))
    return out_pad[:N_NODES, :NK]
